# baseline (speedup 1.0000x reference)
import functools

import numpy as np
import jax
import jax.numpy as jnp
from jax import lax
from jax.experimental import pallas as pl
from jax.experimental.pallas import tpu as pltpu
from jax.experimental.pallas import tpu_sc as plsc

F32 = jnp.float32


_CLS_ROWS = 4096


def _cls_body(x_ref, w1, b1, w2, b2, w3, b3, w4, b4, w5, b5, lwt, lb, out_ref):
    h = jnp.dot(x_ref[...], w1[...], preferred_element_type=F32) + b1[...]
    h = jnp.where(h >= 0, h, 0.1 * h)
    h = jnp.dot(h, w2[...], preferred_element_type=F32) + b2[...]
    h = jnp.where(h >= 0, h, 0.1 * h)
    h = jnp.dot(h, w3[...], preferred_element_type=F32) + b3[...]
    h = jnp.where(h >= 0, h, 0.1 * h)
    h = jnp.dot(h, w4[...], preferred_element_type=F32) + b4[...]
    h = jnp.where(h >= 0, h, 0.1 * h)
    h = jnp.dot(h, w5[...], preferred_element_type=F32) + b5[...]
    g = _CLS_ROWS // 64
    m = jnp.mean(h.reshape(g, 64, 32), axis=1)
    out_ref[...] = jnp.dot(m, lwt[...], preferred_element_type=F32) + lb[...]


def _classifier(x, cls):
    B = x.shape[0]
    xc = x.reshape(B, 3, 8, 4, 8, 4).transpose(0, 2, 4, 1, 3, 5).reshape(B * 64, 48)
    w1 = cls['c1w'].reshape(128, 48).T
    w2 = cls['c2w'].reshape(128, 128).T
    w3 = cls['c3w'].reshape(128, 128).T
    w4 = cls['c4w'].reshape(128, 128).T
    w5 = cls['c5w'].reshape(32, 128).T
    lwt = cls['lw'].T
    args = (xc,
            w1, cls['c1b'].reshape(1, 128), w2, cls['c2b'].reshape(1, 128),
            w3, cls['c3b'].reshape(1, 128), w4, cls['c4b'].reshape(1, 128),
            w5, cls['c5b'].reshape(1, 32), lwt, cls['lb'].reshape(1, 3))
    n_steps = (B * 64) // _CLS_ROWS
    g = _CLS_ROWS // 64

    def full(a):
        return pl.BlockSpec(a.shape, lambda i: (0,) * a.ndim)

    in_specs = [pl.BlockSpec((_CLS_ROWS, 48), lambda i: (i, 0))]
    in_specs += [full(a) for a in args[1:]]
    return pl.pallas_call(
        _cls_body,
        grid=(n_steps,),
        in_specs=in_specs,
        out_specs=pl.BlockSpec((g, 3), lambda i: (i, 0)),
        out_shape=jax.ShapeDtypeStruct((B, 3), F32),
    )(*args)


def _route(l0, l1, l2):
    n = l0.shape[0]
    mesh = plsc.VectorSubcoreMesh(core_axis_name="c", subcore_axis_name="s")

    @functools.partial(
        pl.kernel, mesh=mesh,
        out_type=[jax.ShapeDtypeStruct((n,), jnp.int32),
                  jax.ShapeDtypeStruct((48,), jnp.int32)],
        scratch_types=[pltpu.VMEM((n,), F32),
                       pltpu.VMEM((n,), F32),
                       pltpu.VMEM((n,), F32),
                       pltpu.VMEM((n,), jnp.int32),
                       pltpu.VMEM((48,), jnp.int32)],
    )
    def k(l0_hbm, l1_hbm, l2_hbm, idx_hbm, cnt_hbm, v0, v1, v2, vidx, vcnt):
        @pl.when((lax.axis_index("c") == 0) & (lax.axis_index("s") == 0))
        def _():
            pltpu.sync_copy(l0_hbm, v0)
            pltpu.sync_copy(l1_hbm, v1)
            pltpu.sync_copy(l2_hbm, v2)

            def body(t, carry):
                c0, c1, c2 = carry
                a0 = v0[pl.ds(t * 16, 16)]
                a1 = v1[pl.ds(t * 16, 16)]
                a2 = v2[pl.ds(t * 16, 16)]
                one = jnp.ones((16,), jnp.int32)
                zero = jnp.zeros((16,), jnp.int32)
                e = jnp.where((a0 >= a1) & (a0 >= a2), 0,
                              jnp.where(a1 >= a2, 1, 2))
                vidx[pl.ds(t * 16, 16)] = e
                c0 = c0 + jnp.where(e == 0, one, zero)
                c1 = c1 + jnp.where(e == 1, one, zero)
                c2 = c2 + jnp.where(e == 2, one, zero)
                return c0, c1, c2

            z = jnp.zeros((16,), jnp.int32)
            c0, c1, c2 = lax.fori_loop(0, n // 16, body, (z, z, z))
            vcnt[pl.ds(0, 16)] = c0
            vcnt[pl.ds(16, 16)] = c1
            vcnt[pl.ds(32, 16)] = c2
            pltpu.sync_copy(vidx, idx_hbm)
            pltpu.sync_copy(vcnt, cnt_hbm)

    return k(l0, l1, l2)


_G = 4


_W = 36
_NPIX = _W * _W


def _np_padmask():
    m = np.zeros((1, _NPIX), np.float32)
    for i in range(2, 34):
        m[0, i * _W + 2:i * _W + 34] = 1.0
    return m


def _im2col(xb, k, pad):
    cols = []
    for dy in range(k):
        for dx in range(k):
            shift = _W * (dy - pad) + (dx - pad)
            cols.append(jnp.roll(xb, -shift, axis=1) if shift else xb)
    return jnp.concatenate(cols, axis=0)


def _prelu(h, a):
    return jnp.where(h >= 0, h, a * h)


BF16 = jnp.bfloat16


def _branch_net(xc5, pm, refs, out_ref, g):
    (wh, hb, ha, ws, sb, sa, wm, mb, ma, we, eb, ea, wt, tb) = refs
    h = (jnp.dot(wh[...], xc5, preferred_element_type=F32) + hb[...]) * pm
    h = _prelu(h, ha[...])
    h = (jnp.dot(ws[...], h, preferred_element_type=F32) + sb[...]) * pm
    h = _prelu(h, sa[...])
    for kk in range(4):
        xc = _im2col(h.astype(BF16), 3, 1)
        h = (jnp.dot(wm[kk], xc, preferred_element_type=F32) + mb[kk]) * pm
    h = _prelu(h, ma[...])
    h = (jnp.dot(we[...], h, preferred_element_type=F32) + eb[...]) * pm
    h = _prelu(h, ea[...])
    xc = _im2col(h.astype(BF16), 3, 1)
    out_ref[g] = jnp.dot(wt[...], xc, preferred_element_type=F32) + tb[...]


def _expert_body(idx_ref, x_ref, pm_ref, *args):
    refs, out_ref = args[:-1], args[-1]
    i = pl.program_id(0)
    pm = pm_ref[...]
    for g in range(_G):
        e = idx_ref[i * _G + g]
        xb = x_ref[g]
        xc5 = _im2col(xb, 5, 2)
        for de in range(3):
            @pl.when(e == de)
            def _(de=de):
                _branch_net(xc5, pm, refs[de * 14:(de + 1) * 14], out_ref, g)


def _make_wt_index(d):
    zslot = d * 3 * 81
    idxm = np.full((48, 9 * d), zslot, np.int64)
    for a in range(4):
        for ty in range(3):
            ky = 1 - a + 4 * ty
            if not 0 <= ky <= 8:
                continue
            for b in range(4):
                for tx in range(3):
                    kx = 1 - b + 4 * tx
                    if not 0 <= kx <= 8:
                        continue
                    for co in range(3):
                        r = (a * 4 + b) * 3 + co
                        for c in range(d):
                            idxm[r, (ty * 3 + tx) * d + c] = (
                                (c * 3 + co) * 9 + (8 - ky)) * 9 + (8 - kx)
    return idxm


_WT_IDX = {d: _make_wt_index(d) for d in (16, 36, 56)}


def _expert_weights(p):
    d = p['hw'].shape[0]
    wh = p['hw'].transpose(0, 2, 3, 1).reshape(d, 75)
    hb = p['hb'].reshape(d, 1)
    ha = p['ha'].reshape(d, 1)
    ws = p['sw'].reshape(12, d)
    sb = p['sb'].reshape(12, 1)
    sa = p['sa'].reshape(12, 1)
    wm = jnp.stack([p['mw%d' % i].transpose(0, 2, 3, 1).reshape(12, 108)
                    for i in range(4)]).astype(BF16)
    mb = jnp.stack([p['mb%d' % i].reshape(12, 1) for i in range(4)])
    ma = p['ma'].reshape(12, 1)
    we = p['ew'].reshape(d, 12)
    eb = p['eb'].reshape(d, 1)
    ea = p['ea'].reshape(d, 1)
    flat = jnp.concatenate([p['tw'].reshape(-1), jnp.zeros((1,), F32)])
    wt = flat[_WT_IDX[d]].astype(BF16)
    tb = jnp.tile(p['tb'], 16).reshape(48, 1)
    return (wh, hb, ha, ws, sb, sa, wm, mb, ma, we, eb, ea, wt, tb)


def _experts(x, params, idx):
    B = x.shape[0]
    xp = jnp.pad(x, ((0, 0), (0, 0), (2, 2), (2, 2))).reshape(B, 3, _NPIX)
    pm = jnp.asarray(_np_padmask())
    flat_w = []
    for n in ('net1', 'net2', 'net3'):
        flat_w.extend(_expert_weights(params[n]))

    def full(a):
        return pl.BlockSpec(a.shape, lambda i, s: (0,) * a.ndim)

    grid_spec = pltpu.PrefetchScalarGridSpec(
        num_scalar_prefetch=1,
        grid=(B // _G,),
        in_specs=[pl.BlockSpec((_G, 3, _NPIX), lambda i, s: (i, 0, 0)),
                  full(pm)] + [full(a) for a in flat_w],
        out_specs=pl.BlockSpec((_G, 48, _NPIX), lambda i, s: (i, 0, 0)),
    )
    y = pl.pallas_call(
        _expert_body,
        grid_spec=grid_spec,
        out_shape=jax.ShapeDtypeStruct((B, 48, _NPIX), F32),
    )(idx, xp, pm, *flat_w)
    y = y.reshape(B, 48, _W, _W)[:, :, 2:34, 2:34]
    out = y.reshape(B, 4, 4, 3, 32, 32).transpose(0, 3, 4, 1, 5, 2)
    return out.reshape(B, 3, 128, 128)


def kernel(x, params):
    B = x.shape[0]
    logits = _classifier(x, params['cls'])
    l0 = logits[:, 0]
    l1 = logits[:, 1]
    l2 = logits[:, 2]
    idx, cnt48 = _route(l0, l1, l2)
    counts = cnt48.reshape(3, 16).sum(axis=1).astype(jnp.int32)
    out = _experts(x, params, idx)
    return out, counts


# speedup vs baseline: 1.2446x; 1.2446x over previous
import functools

import numpy as np
import jax
import jax.numpy as jnp
from jax import lax
from jax.experimental import pallas as pl
from jax.experimental.pallas import tpu as pltpu
from jax.experimental.pallas import tpu_sc as plsc

F32 = jnp.float32


_CLS_ROWS = 4096


def _cls_body(x_ref, w1, b1, w2, b2, w3, b3, w4, b4, w5, b5, lwt, lb, out_ref):
    h = jnp.dot(x_ref[...], w1[...], preferred_element_type=F32) + b1[...]
    h = jnp.where(h >= 0, h, 0.1 * h)
    h = jnp.dot(h, w2[...], preferred_element_type=F32) + b2[...]
    h = jnp.where(h >= 0, h, 0.1 * h)
    h = jnp.dot(h, w3[...], preferred_element_type=F32) + b3[...]
    h = jnp.where(h >= 0, h, 0.1 * h)
    h = jnp.dot(h, w4[...], preferred_element_type=F32) + b4[...]
    h = jnp.where(h >= 0, h, 0.1 * h)
    h = jnp.dot(h, w5[...], preferred_element_type=F32) + b5[...]
    g = _CLS_ROWS // 64
    m = jnp.mean(h.reshape(g, 64, 32), axis=1)
    out_ref[...] = jnp.dot(m, lwt[...], preferred_element_type=F32) + lb[...]


def _classifier(x, cls):
    B = x.shape[0]
    xc = x.reshape(B, 3, 8, 4, 8, 4).transpose(0, 2, 4, 1, 3, 5).reshape(B * 64, 48)
    w1 = cls['c1w'].reshape(128, 48).T
    w2 = cls['c2w'].reshape(128, 128).T
    w3 = cls['c3w'].reshape(128, 128).T
    w4 = cls['c4w'].reshape(128, 128).T
    w5 = cls['c5w'].reshape(32, 128).T
    lwt = cls['lw'].T
    args = (xc,
            w1, cls['c1b'].reshape(1, 128), w2, cls['c2b'].reshape(1, 128),
            w3, cls['c3b'].reshape(1, 128), w4, cls['c4b'].reshape(1, 128),
            w5, cls['c5b'].reshape(1, 32), lwt, cls['lb'].reshape(1, 3))
    n_steps = (B * 64) // _CLS_ROWS
    g = _CLS_ROWS // 64

    def full(a):
        return pl.BlockSpec(a.shape, lambda i: (0,) * a.ndim)

    in_specs = [pl.BlockSpec((_CLS_ROWS, 48), lambda i: (i, 0))]
    in_specs += [full(a) for a in args[1:]]
    return pl.pallas_call(
        _cls_body,
        grid=(n_steps,),
        in_specs=in_specs,
        out_specs=pl.BlockSpec((g, 3), lambda i: (i, 0)),
        out_shape=jax.ShapeDtypeStruct((B, 3), F32),
    )(*args)


def _route(l0, l1, l2):
    n = l0.shape[0]
    mesh = plsc.VectorSubcoreMesh(core_axis_name="c", subcore_axis_name="s")

    @functools.partial(
        pl.kernel, mesh=mesh,
        out_type=[jax.ShapeDtypeStruct((n,), jnp.int32),
                  jax.ShapeDtypeStruct((48,), jnp.int32)],
        scratch_types=[pltpu.VMEM((n,), F32),
                       pltpu.VMEM((n,), F32),
                       pltpu.VMEM((n,), F32),
                       pltpu.VMEM((n,), jnp.int32),
                       pltpu.VMEM((48,), jnp.int32)],
    )
    def k(l0_hbm, l1_hbm, l2_hbm, idx_hbm, cnt_hbm, v0, v1, v2, vidx, vcnt):
        @pl.when((lax.axis_index("c") == 0) & (lax.axis_index("s") == 0))
        def _():
            pltpu.sync_copy(l0_hbm, v0)
            pltpu.sync_copy(l1_hbm, v1)
            pltpu.sync_copy(l2_hbm, v2)

            def body(t, carry):
                c0, c1, c2 = carry
                a0 = v0[pl.ds(t * 16, 16)]
                a1 = v1[pl.ds(t * 16, 16)]
                a2 = v2[pl.ds(t * 16, 16)]
                one = jnp.ones((16,), jnp.int32)
                zero = jnp.zeros((16,), jnp.int32)
                e = jnp.where((a0 >= a1) & (a0 >= a2), 0,
                              jnp.where(a1 >= a2, 1, 2))
                vidx[pl.ds(t * 16, 16)] = e
                c0 = c0 + jnp.where(e == 0, one, zero)
                c1 = c1 + jnp.where(e == 1, one, zero)
                c2 = c2 + jnp.where(e == 2, one, zero)
                return c0, c1, c2

            z = jnp.zeros((16,), jnp.int32)
            c0, c1, c2 = lax.fori_loop(0, n // 16, body, (z, z, z))
            vcnt[pl.ds(0, 16)] = c0
            vcnt[pl.ds(16, 16)] = c1
            vcnt[pl.ds(32, 16)] = c2
            pltpu.sync_copy(vidx, idx_hbm)
            pltpu.sync_copy(vcnt, cnt_hbm)

    return k(l0, l1, l2)


_G = 8
_W = 34
_NPIX = _W * _W
_NAL = 1280


def _np_padmask():
    m = np.zeros((1, _NAL), np.float32)
    for i in range(1, 33):
        m[0, i * _W + 1:i * _W + 33] = 1.0
    return np.tile(m, (1, _G))


def _im2col(xb, k, pad):
    cols = []
    for dy in range(k):
        for dx in range(k):
            shift = _W * (dy - pad) + (dx - pad)
            cols.append(jnp.roll(xb, -shift, axis=1) if shift else xb)
    return jnp.concatenate(cols, axis=0)


def _prelu(h, a):
    return jnp.where(h >= 0, h, a * h)


BF16 = jnp.bfloat16


def _branch_net(xc5, pm, refs):
    (wh, hb, ha, ws, sb, sa, wm, mb, ma, we, eb, ea, wt, tb) = refs
    h = (jnp.dot(wh[...], xc5, preferred_element_type=F32) + hb[...]) * pm
    h = _prelu(h, ha[...])
    h = (jnp.dot(ws[...], h, preferred_element_type=F32) + sb[...]) * pm
    h = _prelu(h, sa[...])
    for kk in range(4):
        xc = _im2col(h.astype(BF16), 3, 1)
        h = (jnp.dot(wm[kk], xc, preferred_element_type=F32) + mb[kk]) * pm
    h = _prelu(h, ma[...])
    h = (jnp.dot(we[...], h, preferred_element_type=F32) + eb[...]) * pm
    h = _prelu(h, ea[...])
    xc = _im2col(h.astype(BF16), 3, 1)
    return jnp.dot(wt[...], xc, preferred_element_type=F32) + tb[...]


def _expert_body(idx_ref, x_ref, pm_ref, *args):
    refs, out_ref = args[:-1], args[-1]
    i = pl.program_id(0)
    base = i * _G
    pm = pm_ref[...]
    xb = x_ref[0]
    xc5 = _im2col(xb, 5, 2)
    for de in range(3):
        cond = idx_ref[base] == de
        for g in range(1, _G):
            cond = cond | (idx_ref[base + g] == de)

        @pl.when(cond)
        def _(de=de):
            y = _branch_net(xc5, pm, refs[de * 14:(de + 1) * 14])
            for g in range(_G):
                @pl.when(idx_ref[base + g] == de)
                def _(g=g, y=y):
                    out_ref[g] = y[:, g * _NAL:(g + 1) * _NAL]


def _make_wt_index(d):
    zslot = d * 3 * 81
    idxm = np.full((48, 9 * d), zslot, np.int64)
    for a in range(4):
        for ty in range(3):
            ky = 1 - a + 4 * ty
            if not 0 <= ky <= 8:
                continue
            for b in range(4):
                for tx in range(3):
                    kx = 1 - b + 4 * tx
                    if not 0 <= kx <= 8:
                        continue
                    for co in range(3):
                        r = (a * 4 + b) * 3 + co
                        for c in range(d):
                            idxm[r, (ty * 3 + tx) * d + c] = (
                                (c * 3 + co) * 9 + (8 - ky)) * 9 + (8 - kx)
    return idxm


_WT_IDX = {d: _make_wt_index(d) for d in (16, 36, 56)}


def _expert_weights(p):
    d = p['hw'].shape[0]
    wh = p['hw'].transpose(0, 2, 3, 1).reshape(d, 75)
    hb = p['hb'].reshape(d, 1)
    ha = p['ha'].reshape(d, 1)
    ws = p['sw'].reshape(12, d)
    sb = p['sb'].reshape(12, 1)
    sa = p['sa'].reshape(12, 1)
    wm = jnp.stack([p['mw%d' % i].transpose(0, 2, 3, 1).reshape(12, 108)
                    for i in range(4)]).astype(BF16)
    mb = jnp.stack([p['mb%d' % i].reshape(12, 1) for i in range(4)])
    ma = p['ma'].reshape(12, 1)
    we = p['ew'].reshape(d, 12)
    eb = p['eb'].reshape(d, 1)
    ea = p['ea'].reshape(d, 1)
    flat = jnp.concatenate([p['tw'].reshape(-1), jnp.zeros((1,), F32)])
    wt = flat[_WT_IDX[d]].astype(BF16)
    tb = jnp.tile(p['tb'], 16).reshape(48, 1)
    return (wh, hb, ha, ws, sb, sa, wm, mb, ma, we, eb, ea, wt, tb)


def _experts(x, params, idx):
    B = x.shape[0]
    perm = jnp.argsort(idx, stable=True)
    inv = jnp.argsort(perm, stable=True)
    idx_s = idx[perm]
    xp = jnp.pad(x, ((0, 0), (0, 0), (1, 1), (1, 1))).reshape(B, 3, _NPIX)
    xp = jnp.pad(xp, ((0, 0), (0, 0), (0, _NAL - _NPIX)))
    xs = (xp[perm].reshape(B // _G, _G, 3, _NAL).transpose(0, 2, 1, 3)
          .reshape(B // _G, 3, _G * _NAL))
    pm = jnp.asarray(_np_padmask())
    flat_w = []
    for n in ('net1', 'net2', 'net3'):
        flat_w.extend(_expert_weights(params[n]))

    def full(a):
        return pl.BlockSpec(a.shape, lambda i, s: (0,) * a.ndim)

    grid_spec = pltpu.PrefetchScalarGridSpec(
        num_scalar_prefetch=1,
        grid=(B // _G,),
        in_specs=[pl.BlockSpec((1, 3, _G * _NAL), lambda i, s: (i, 0, 0)),
                  full(pm)] + [full(a) for a in flat_w],
        out_specs=pl.BlockSpec((_G, 48, _NAL), lambda i, s: (i, 0, 0)),
    )
    y = pl.pallas_call(
        _expert_body,
        grid_spec=grid_spec,
        out_shape=jax.ShapeDtypeStruct((B, 48, _NAL), F32),
    )(idx_s, xs, pm, *flat_w)
    y = y[inv][:, :, :_NPIX].reshape(B, 48, _W, _W)[:, :, 1:33, 1:33]
    out = y.reshape(B, 4, 4, 3, 32, 32).transpose(0, 3, 4, 1, 5, 2)
    return out.reshape(B, 3, 128, 128)


def kernel(x, params):
    B = x.shape[0]
    logits = _classifier(x, params['cls'])
    l0 = logits[:, 0]
    l1 = logits[:, 1]
    l2 = logits[:, 2]
    idx, cnt48 = _route(l0, l1, l2)
    counts = cnt48.reshape(3, 16).sum(axis=1).astype(jnp.int32)
    out = _experts(x, params, idx)
    return out, counts


# speedup vs baseline: 1.3229x; 1.0629x over previous
import functools

import numpy as np
import jax
import jax.numpy as jnp
from jax import lax
from jax.experimental import pallas as pl
from jax.experimental.pallas import tpu as pltpu
from jax.experimental.pallas import tpu_sc as plsc

F32 = jnp.float32


_CLS_ROWS = 4096


def _cls_body(x_ref, w1, b1, w2, b2, w3, b3, w4, b4, w5, b5, lwt, lb, out_ref):
    h = jnp.dot(x_ref[...], w1[...], preferred_element_type=F32) + b1[...]
    h = jnp.where(h >= 0, h, 0.1 * h)
    h = jnp.dot(h, w2[...], preferred_element_type=F32) + b2[...]
    h = jnp.where(h >= 0, h, 0.1 * h)
    h = jnp.dot(h, w3[...], preferred_element_type=F32) + b3[...]
    h = jnp.where(h >= 0, h, 0.1 * h)
    h = jnp.dot(h, w4[...], preferred_element_type=F32) + b4[...]
    h = jnp.where(h >= 0, h, 0.1 * h)
    h = jnp.dot(h, w5[...], preferred_element_type=F32) + b5[...]
    g = _CLS_ROWS // 64
    m = jnp.mean(h.reshape(g, 64, 32), axis=1)
    out_ref[...] = jnp.dot(m, lwt[...], preferred_element_type=F32) + lb[...]


def _classifier(x, cls):
    B = x.shape[0]
    xc = x.reshape(B, 3, 8, 4, 8, 4).transpose(0, 2, 4, 1, 3, 5).reshape(B * 64, 48)
    w1 = cls['c1w'].reshape(128, 48).T
    w2 = cls['c2w'].reshape(128, 128).T
    w3 = cls['c3w'].reshape(128, 128).T
    w4 = cls['c4w'].reshape(128, 128).T
    w5 = cls['c5w'].reshape(32, 128).T
    lwt = cls['lw'].T
    args = (xc,
            w1, cls['c1b'].reshape(1, 128), w2, cls['c2b'].reshape(1, 128),
            w3, cls['c3b'].reshape(1, 128), w4, cls['c4b'].reshape(1, 128),
            w5, cls['c5b'].reshape(1, 32), lwt, cls['lb'].reshape(1, 3))
    n_steps = (B * 64) // _CLS_ROWS
    g = _CLS_ROWS // 64

    def full(a):
        return pl.BlockSpec(a.shape, lambda i: (0,) * a.ndim)

    in_specs = [pl.BlockSpec((_CLS_ROWS, 48), lambda i: (i, 0))]
    in_specs += [full(a) for a in args[1:]]
    return pl.pallas_call(
        _cls_body,
        grid=(n_steps,),
        in_specs=in_specs,
        out_specs=pl.BlockSpec((g, 3), lambda i: (i, 0)),
        out_shape=jax.ShapeDtypeStruct((B, 3), F32),
    )(*args)


def _route(l0, l1, l2):
    n = l0.shape[0]
    mesh = plsc.VectorSubcoreMesh(core_axis_name="c", subcore_axis_name="s")

    @functools.partial(
        pl.kernel, mesh=mesh,
        out_type=[jax.ShapeDtypeStruct((n,), jnp.int32),
                  jax.ShapeDtypeStruct((48,), jnp.int32)],
        scratch_types=[pltpu.VMEM((n,), F32),
                       pltpu.VMEM((n,), F32),
                       pltpu.VMEM((n,), F32),
                       pltpu.VMEM((n,), jnp.int32),
                       pltpu.VMEM((48,), jnp.int32)],
    )
    def k(l0_hbm, l1_hbm, l2_hbm, idx_hbm, cnt_hbm, v0, v1, v2, vidx, vcnt):
        @pl.when((lax.axis_index("c") == 0) & (lax.axis_index("s") == 0))
        def _():
            pltpu.sync_copy(l0_hbm, v0)
            pltpu.sync_copy(l1_hbm, v1)
            pltpu.sync_copy(l2_hbm, v2)

            def body(t, carry):
                c0, c1, c2 = carry
                a0 = v0[pl.ds(t * 16, 16)]
                a1 = v1[pl.ds(t * 16, 16)]
                a2 = v2[pl.ds(t * 16, 16)]
                one = jnp.ones((16,), jnp.int32)
                zero = jnp.zeros((16,), jnp.int32)
                e = jnp.where((a0 >= a1) & (a0 >= a2), 0,
                              jnp.where(a1 >= a2, 1, 2))
                vidx[pl.ds(t * 16, 16)] = e
                c0 = c0 + jnp.where(e == 0, one, zero)
                c1 = c1 + jnp.where(e == 1, one, zero)
                c2 = c2 + jnp.where(e == 2, one, zero)
                return c0, c1, c2

            z = jnp.zeros((16,), jnp.int32)
            c0, c1, c2 = lax.fori_loop(0, n // 16, body, (z, z, z))
            vcnt[pl.ds(0, 16)] = c0
            vcnt[pl.ds(16, 16)] = c1
            vcnt[pl.ds(32, 16)] = c2
            pltpu.sync_copy(vidx, idx_hbm)
            pltpu.sync_copy(vcnt, cnt_hbm)

    return k(l0, l1, l2)


_G = 8
_W = 34
_NPIX = _W * _W
_NAL = 1280


def _np_padmask():
    m = np.zeros((1, _NAL), np.float32)
    for i in range(1, 33):
        m[0, i * _W + 1:i * _W + 33] = 1.0
    return np.tile(m, (1, _G))


def _im2col(xb, k, pad):
    cols = []
    for dy in range(k):
        for dx in range(k):
            shift = _W * (dy - pad) + (dx - pad)
            cols.append(jnp.roll(xb, -shift, axis=1) if shift else xb)
    return jnp.concatenate(cols, axis=0)


def _prelu(h, a):
    return jnp.where(h >= 0, h, a * h)


BF16 = jnp.bfloat16


def _branch_net(xc5, pm, refs):
    (wh, hb, ha, ws, sb, sa, wm, mb, ma, we, eb, ea, wt, tb) = refs
    h = (jnp.dot(wh[...], xc5, preferred_element_type=F32) + hb[...]) * pm
    h = _prelu(h, ha[...])
    h = (jnp.dot(ws[...], h, preferred_element_type=F32) + sb[...]) * pm
    h = _prelu(h, sa[...])
    for kk in range(4):
        xc = _im2col(h.astype(BF16), 3, 1)
        h = (jnp.dot(wm[kk], xc, preferred_element_type=F32) + mb[kk]) * pm
    h = _prelu(h, ma[...])
    h = (jnp.dot(we[...], h, preferred_element_type=F32) + eb[...]) * pm
    h = _prelu(h, ea[...])
    xc = _im2col(h.astype(BF16), 3, 1)
    y = jnp.dot(wt[...], xc, preferred_element_type=F32) + tb[...]
    return y.astype(BF16)


def _expert_body(idx_ref, x_ref, pm_ref, *args):
    refs, out_ref = args[:-1], args[-1]
    i = pl.program_id(0)
    base = i * _G
    pm = pm_ref[...]
    xb = x_ref[0]
    xc5 = _im2col(xb, 5, 2)
    for de in range(3):
        cond = idx_ref[base] == de
        for g in range(1, _G):
            cond = cond | (idx_ref[base + g] == de)

        @pl.when(cond)
        def _(de=de):
            y = _branch_net(xc5, pm, refs[de * 14:(de + 1) * 14])
            for g in range(_G):
                @pl.when(idx_ref[base + g] == de)
                def _(g=g, y=y):
                    out_ref[g] = y[:, g * _NAL:(g + 1) * _NAL]


def _make_wt_index(d):
    zslot = d * 3 * 81
    idxm = np.full((48, 9 * d), zslot, np.int64)
    for a in range(4):
        for ty in range(3):
            ky = 1 - a + 4 * ty
            if not 0 <= ky <= 8:
                continue
            for b in range(4):
                for tx in range(3):
                    kx = 1 - b + 4 * tx
                    if not 0 <= kx <= 8:
                        continue
                    for co in range(3):
                        r = (a * 4 + b) * 3 + co
                        for c in range(d):
                            idxm[r, (ty * 3 + tx) * d + c] = (
                                (c * 3 + co) * 9 + (8 - ky)) * 9 + (8 - kx)
    return idxm


_WT_IDX = {d: _make_wt_index(d) for d in (16, 36, 56)}


def _expert_weights(p):
    d = p['hw'].shape[0]
    wh = p['hw'].transpose(0, 2, 3, 1).reshape(d, 75)
    hb = p['hb'].reshape(d, 1)
    ha = p['ha'].reshape(d, 1)
    ws = p['sw'].reshape(12, d)
    sb = p['sb'].reshape(12, 1)
    sa = p['sa'].reshape(12, 1)
    wm = jnp.stack([p['mw%d' % i].transpose(0, 2, 3, 1).reshape(12, 108)
                    for i in range(4)]).astype(BF16)
    mb = jnp.stack([p['mb%d' % i].reshape(12, 1) for i in range(4)])
    ma = p['ma'].reshape(12, 1)
    we = p['ew'].reshape(d, 12)
    eb = p['eb'].reshape(d, 1)
    ea = p['ea'].reshape(d, 1)
    flat = jnp.concatenate([p['tw'].reshape(-1), jnp.zeros((1,), F32)])
    wt = flat[_WT_IDX[d]].astype(BF16)
    tb = jnp.tile(p['tb'], 16).reshape(48, 1)
    return (wh, hb, ha, ws, sb, sa, wm, mb, ma, we, eb, ea, wt, tb)


def _experts(x, params, idx):
    B = x.shape[0]
    perm = jnp.argsort(idx, stable=True)
    inv = jnp.argsort(perm, stable=True)
    idx_s = idx[perm]
    xp = jnp.pad(x, ((0, 0), (0, 0), (1, 1), (1, 1))).reshape(B, 3, _NPIX)
    xp = jnp.pad(xp, ((0, 0), (0, 0), (0, _NAL - _NPIX)))
    xs = (xp[perm].reshape(B // _G, _G, 3, _NAL).transpose(0, 2, 1, 3)
          .reshape(B // _G, 3, _G * _NAL))
    pm = jnp.asarray(_np_padmask())
    flat_w = []
    for n in ('net1', 'net2', 'net3'):
        flat_w.extend(_expert_weights(params[n]))

    def full(a):
        return pl.BlockSpec(a.shape, lambda i, s: (0,) * a.ndim)

    grid_spec = pltpu.PrefetchScalarGridSpec(
        num_scalar_prefetch=1,
        grid=(B // _G,),
        in_specs=[pl.BlockSpec((1, 3, _G * _NAL), lambda i, s: (i, 0, 0)),
                  full(pm)] + [full(a) for a in flat_w],
        out_specs=pl.BlockSpec((_G, 48, _NAL), lambda i, s: (i, 0, 0)),
    )
    y = pl.pallas_call(
        _expert_body,
        grid_spec=grid_spec,
        out_shape=jax.ShapeDtypeStruct((B, 48, _NAL), BF16),
    )(idx_s, xs, pm, *flat_w)
    y = y[inv][:, :, :_NPIX].reshape(B, 48, _W, _W)[:, :, 1:33, 1:33]
    out = y.reshape(B, 4, 4, 3, 32, 32).transpose(0, 3, 4, 1, 5, 2)
    return out.reshape(B, 3, 128, 128).astype(F32)


def kernel(x, params):
    B = x.shape[0]
    logits = _classifier(x, params['cls'])
    l0 = logits[:, 0]
    l1 = logits[:, 1]
    l2 = logits[:, 2]
    idx, cnt48 = _route(l0, l1, l2)
    counts = cnt48.reshape(3, 16).sum(axis=1).astype(jnp.int32)
    out = _experts(x, params, idx)
    return out, counts
